# Initial kernel scaffold; baseline (speedup 1.0000x reference)
import jax
import jax.numpy as jnp
from jax.experimental import pallas as pl


def kernel(attn, relative_position_bias_table, relative_position_index):
    raise NotImplementedError("write your pallas kernel here")


# speedup vs baseline: 5.4906x; 5.4906x over previous
import functools

import jax
import jax.numpy as jnp
from jax import lax
from jax.experimental import pallas as pl
from jax.experimental.pallas import tpu as pltpu
from jax.experimental.pallas import tpu_sc as plsc

_NUM_CORES = 2
_NUM_SUBCORES = 16
_NUM_WORKERS = _NUM_CORES * _NUM_SUBCORES
_LANES = 16


def _sc_gather_bias(table_flat, idx_flat, num_channels):
    n_pos = idx_flat.shape[0]
    per_worker = n_pos // _NUM_WORKERS
    half = per_worker // 2
    tab_words = table_flat.shape[0]
    mesh = plsc.VectorSubcoreMesh(core_axis_name="c", subcore_axis_name="s")

    @functools.partial(
        pl.kernel,
        mesh=mesh,
        compiler_params=pltpu.CompilerParams(needs_layout_passes=False),
        out_type=jax.ShapeDtypeStruct((num_channels, n_pos), jnp.float32),
        scratch_types=[
            pltpu.VMEM((per_worker,), jnp.int32),
            pltpu.VMEM((tab_words,), jnp.float32),
            pltpu.VMEM((num_channels, half), jnp.float32),
        ],
    )
    def gather_kernel(tab_hbm, idx_hbm, bias_hbm, idx_v, tab_v, out_v):
        wid = lax.axis_index("s") * _NUM_CORES + lax.axis_index("c")
        base = wid * per_worker
        pltpu.sync_copy(idx_hbm.at[pl.ds(base, per_worker)], idx_v)
        pltpu.sync_copy(tab_hbm, tab_v)
        for h in range(2):
            off = h * half

            def body(k, _, off=off):
                k16 = k * _LANES
                iv = idx_v[pl.ds(off + k16, _LANES)] * num_channels
                for c in range(num_channels):
                    out_v[c, pl.ds(k16, _LANES)] = plsc.load_gather(
                        tab_v, [iv + c]
                    )
                return 0

            lax.fori_loop(0, half // _LANES, body, 0)
            for c in range(num_channels):
                pltpu.sync_copy(
                    out_v.at[c], bias_hbm.at[c, pl.ds(base + off, half)]
                )

    return gather_kernel(table_flat, idx_flat)


def _tc_add(attn, bias):
    batch, channels, n, n_sr = attn.shape

    def add_body(a_ref, b_ref, o_ref):
        o_ref[...] = a_ref[...] + b_ref[...]

    return pl.pallas_call(
        add_body,
        grid=(channels, batch),
        in_specs=[
            pl.BlockSpec((1, 1, n, n_sr), lambda c, b: (b, c, 0, 0)),
            pl.BlockSpec((1, n, n_sr), lambda c, b: (c, 0, 0)),
        ],
        out_specs=pl.BlockSpec((1, 1, n, n_sr), lambda c, b: (b, c, 0, 0)),
        out_shape=jax.ShapeDtypeStruct(attn.shape, attn.dtype),
    )(attn, bias)


def kernel(attn, relative_position_bias_table, relative_position_index):
    batch, channels, n, n_sr = attn.shape
    table_flat = relative_position_bias_table.reshape(-1)
    idx_flat = relative_position_index.reshape(-1).astype(jnp.int32)
    bias = _sc_gather_bias(table_flat, idx_flat, channels)
    bias = bias.reshape(channels, n, n_sr)
    return _tc_add(attn, bias)


# speedup vs baseline: 7.5355x; 1.3724x over previous
import functools

import jax
import jax.numpy as jnp
from jax import lax
from jax.experimental import pallas as pl
from jax.experimental.pallas import tpu as pltpu
from jax.experimental.pallas import tpu_sc as plsc

_NUM_CORES = 2
_NUM_SUBCORES = 16
_NUM_WORKERS = _NUM_CORES * _NUM_SUBCORES
_LANES = 16


def _sc_gather_bias(table_flat, idx_flat, num_channels, n, n_sr):
    rows_per_worker = n // _NUM_WORKERS
    row_chunk = rows_per_worker // 2
    half_sr = n_sr // 2
    groups = n_sr // _LANES
    tab_words = table_flat.shape[0]
    mesh = plsc.VectorSubcoreMesh(core_axis_name="c", subcore_axis_name="s")

    plane = jax.ShapeDtypeStruct((num_channels, n, half_sr), jnp.float32)

    @functools.partial(
        pl.kernel,
        mesh=mesh,
        compiler_params=pltpu.CompilerParams(needs_layout_passes=False),
        out_type=(plane, plane),
        scratch_types=[
            pltpu.VMEM((rows_per_worker * n_sr,), jnp.int32),
            pltpu.VMEM((tab_words,), jnp.float32),
            pltpu.VMEM((num_channels, row_chunk, half_sr), jnp.float32),
            pltpu.VMEM((num_channels, row_chunk, half_sr), jnp.float32),
        ],
    )
    def gather_kernel(tab_hbm, idx_hbm, lo_hbm, hi_hbm, idx_v, tab_v, lo_v, hi_v):
        wid = lax.axis_index("s") * _NUM_CORES + lax.axis_index("c")
        row0 = wid * rows_per_worker
        pltpu.sync_copy(idx_hbm.at[pl.ds(row0 * n_sr, rows_per_worker * n_sr)], idx_v)
        pltpu.sync_copy(tab_hbm, tab_v)
        for chunk in range(2):
            crow = chunk * row_chunk

            def row_body(r, _, crow=crow):
                flat = (crow + r) * n_sr
                for k in range(groups):
                    iv = idx_v[pl.ds(flat + k * _LANES, _LANES)] * num_channels
                    dst = lo_v if k < groups // 2 else hi_v
                    col = (k % (groups // 2)) * _LANES
                    for c in range(num_channels):
                        dst[c, r, pl.ds(col, _LANES)] = plsc.load_gather(
                            tab_v, [iv + c]
                        )
                return 0

            lax.fori_loop(0, row_chunk, row_body, 0)
            for c in range(num_channels):
                pltpu.sync_copy(
                    lo_v.at[c], lo_hbm.at[c, pl.ds(row0 + crow, row_chunk)]
                )
                pltpu.sync_copy(
                    hi_v.at[c], hi_hbm.at[c, pl.ds(row0 + crow, row_chunk)]
                )

    return gather_kernel(table_flat, idx_flat)


def _tc_add(attn, bias_lo, bias_hi):
    batch, channels, n, n_sr = attn.shape
    half_sr = n_sr // 2
    c_blk = 2

    def add_body(a_ref, lo_ref, hi_ref, o_ref):
        o_ref[:, :, :, 0:half_sr] = a_ref[:, :, :, 0:half_sr] + lo_ref[...]
        o_ref[:, :, :, half_sr:n_sr] = a_ref[:, :, :, half_sr:n_sr] + hi_ref[...]

    return pl.pallas_call(
        add_body,
        grid=(channels // c_blk, batch),
        in_specs=[
            pl.BlockSpec((1, c_blk, n, n_sr), lambda c, b: (b, c, 0, 0)),
            pl.BlockSpec((c_blk, n, half_sr), lambda c, b: (c, 0, 0)),
            pl.BlockSpec((c_blk, n, half_sr), lambda c, b: (c, 0, 0)),
        ],
        out_specs=pl.BlockSpec((1, c_blk, n, n_sr), lambda c, b: (b, c, 0, 0)),
        out_shape=jax.ShapeDtypeStruct(attn.shape, attn.dtype),
    )(attn, bias_lo, bias_hi)


def kernel(attn, relative_position_bias_table, relative_position_index):
    batch, channels, n, n_sr = attn.shape
    table_flat = relative_position_bias_table.reshape(-1)
    idx_flat = relative_position_index.reshape(-1).astype(jnp.int32)
    bias_lo, bias_hi = _sc_gather_bias(table_flat, idx_flat, channels, n, n_sr)
    return _tc_add(attn, bias_lo, bias_hi)


# speedup vs baseline: 7.6475x; 1.0149x over previous
import functools

import jax
import jax.numpy as jnp
from jax import lax
from jax.experimental import pallas as pl
from jax.experimental.pallas import tpu as pltpu
from jax.experimental.pallas import tpu_sc as plsc

_NUM_CORES = 2
_NUM_SUBCORES = 16
_NUM_WORKERS = _NUM_CORES * _NUM_SUBCORES
_LANES = 16


def _sc_gather_bias(table_flat, idx_flat, num_channels, n, n_sr):
    rows_per_worker = n // _NUM_WORKERS
    row_chunk = rows_per_worker // 2
    half_sr = n_sr // 2
    groups = n_sr // _LANES
    tab_words = table_flat.shape[0]
    mesh = plsc.VectorSubcoreMesh(core_axis_name="c", subcore_axis_name="s")

    plane = jax.ShapeDtypeStruct((num_channels, n, half_sr), jnp.float32)

    @functools.partial(
        pl.kernel,
        mesh=mesh,
        compiler_params=pltpu.CompilerParams(needs_layout_passes=False),
        out_type=(plane, plane),
        scratch_types=[
            pltpu.VMEM((rows_per_worker * n_sr,), jnp.int32),
            pltpu.VMEM((tab_words,), jnp.float32),
            pltpu.VMEM((num_channels, row_chunk, half_sr), jnp.float32),
            pltpu.VMEM((num_channels, row_chunk, half_sr), jnp.float32),
        ],
    )
    def gather_kernel(tab_hbm, idx_hbm, lo_hbm, hi_hbm, idx_v, tab_v, lo_v, hi_v):
        wid = lax.axis_index("s") * _NUM_CORES + lax.axis_index("c")
        row0 = wid * rows_per_worker
        pltpu.sync_copy(idx_hbm.at[pl.ds(row0 * n_sr, rows_per_worker * n_sr)], idx_v)
        pltpu.sync_copy(tab_hbm, tab_v)
        for chunk in range(2):
            crow = chunk * row_chunk

            @plsc.parallel_loop(0, row_chunk, unroll=2)
            def row_body(r, crow=crow):
                flat = (crow + r) * n_sr
                for k in range(groups):
                    iv = idx_v[pl.ds(flat + k * _LANES, _LANES)] * num_channels
                    dst = lo_v if k < groups // 2 else hi_v
                    col = (k % (groups // 2)) * _LANES
                    for c in range(num_channels):
                        dst[c, r, pl.ds(col, _LANES)] = plsc.load_gather(
                            tab_v, [iv + c]
                        )
            for c in range(num_channels):
                pltpu.sync_copy(
                    lo_v.at[c], lo_hbm.at[c, pl.ds(row0 + crow, row_chunk)]
                )
                pltpu.sync_copy(
                    hi_v.at[c], hi_hbm.at[c, pl.ds(row0 + crow, row_chunk)]
                )

    return gather_kernel(table_flat, idx_flat)


def _tc_add(attn, bias_lo, bias_hi):
    batch, channels, n, n_sr = attn.shape
    half_sr = n_sr // 2
    c_blk = 2

    def add_body(a_ref, lo_ref, hi_ref, o_ref):
        o_ref[:, :, :, 0:half_sr] = a_ref[:, :, :, 0:half_sr] + lo_ref[...]
        o_ref[:, :, :, half_sr:n_sr] = a_ref[:, :, :, half_sr:n_sr] + hi_ref[...]

    return pl.pallas_call(
        add_body,
        grid=(channels // c_blk, batch),
        in_specs=[
            pl.BlockSpec((1, c_blk, n, n_sr), lambda c, b: (b, c, 0, 0)),
            pl.BlockSpec((c_blk, n, half_sr), lambda c, b: (c, 0, 0)),
            pl.BlockSpec((c_blk, n, half_sr), lambda c, b: (c, 0, 0)),
        ],
        out_specs=pl.BlockSpec((1, c_blk, n, n_sr), lambda c, b: (b, c, 0, 0)),
        out_shape=jax.ShapeDtypeStruct(attn.shape, attn.dtype),
    )(attn, bias_lo, bias_hi)


def kernel(attn, relative_position_bias_table, relative_position_index):
    batch, channels, n, n_sr = attn.shape
    table_flat = relative_position_bias_table.reshape(-1)
    idx_flat = relative_position_index.reshape(-1).astype(jnp.int32)
    bias_lo, bias_hi = _sc_gather_bias(table_flat, idx_flat, channels, n, n_sr)
    return _tc_add(attn, bias_lo, bias_hi)


# speedup vs baseline: 7.9233x; 1.0361x over previous
import functools

import jax
import jax.numpy as jnp
from jax import lax
from jax.experimental import pallas as pl
from jax.experimental.pallas import tpu as pltpu
from jax.experimental.pallas import tpu_sc as plsc

_NUM_CORES = 2
_NUM_SUBCORES = 16
_NUM_WORKERS = _NUM_CORES * _NUM_SUBCORES
_LANES = 16


def _sc_gather_bias(table_flat, idx_flat, num_channels, n, n_sr):
    rows_per_worker = n // _NUM_WORKERS
    row_chunk = rows_per_worker // 2
    half_sr = n_sr // 2
    groups = n_sr // _LANES
    tab_words = table_flat.shape[0]
    table_rows = tab_words // num_channels
    mesh = plsc.VectorSubcoreMesh(core_axis_name="c", subcore_axis_name="s")

    plane = jax.ShapeDtypeStruct((num_channels, n, half_sr), jnp.float32)

    @functools.partial(
        pl.kernel,
        mesh=mesh,
        compiler_params=pltpu.CompilerParams(needs_layout_passes=False),
        out_type=(plane, plane),
        scratch_types=[
            pltpu.VMEM((rows_per_worker * n_sr,), jnp.int32),
            pltpu.VMEM((tab_words,), jnp.float32),
            pltpu.VMEM((num_channels, row_chunk, half_sr), jnp.float32),
            pltpu.VMEM((num_channels, row_chunk, half_sr), jnp.float32),
        ],
    )
    def gather_kernel(tab_hbm, idx_hbm, lo_hbm, hi_hbm, idx_v, tab_v, lo_v, hi_v):
        wid = lax.axis_index("s") * _NUM_CORES + lax.axis_index("c")
        row0 = wid * rows_per_worker
        pltpu.sync_copy(idx_hbm.at[pl.ds(row0 * n_sr, rows_per_worker * n_sr)], idx_v)
        pltpu.sync_copy(tab_hbm, tab_v)
        for chunk in range(2):
            crow = chunk * row_chunk

            @plsc.parallel_loop(0, row_chunk, unroll=2)
            def row_body(r, crow=crow):
                flat = (crow + r) * n_sr
                for k in range(groups):
                    iv = idx_v[pl.ds(flat + k * _LANES, _LANES)]
                    dst = lo_v if k < groups // 2 else hi_v
                    col = (k % (groups // 2)) * _LANES
                    for c in range(num_channels):
                        dst[c, r, pl.ds(col, _LANES)] = plsc.load_gather(
                            tab_v, [iv + c * table_rows]
                        )
            for c in range(num_channels):
                pltpu.sync_copy(
                    lo_v.at[c], lo_hbm.at[c, pl.ds(row0 + crow, row_chunk)]
                )
                pltpu.sync_copy(
                    hi_v.at[c], hi_hbm.at[c, pl.ds(row0 + crow, row_chunk)]
                )

    return gather_kernel(table_flat, idx_flat)


def _tc_add(attn, bias_lo, bias_hi):
    batch, channels, n, n_sr = attn.shape
    half_sr = n_sr // 2
    c_blk = 2

    def add_body(a_ref, lo_ref, hi_ref, o_ref):
        o_ref[:, :, :, 0:half_sr] = a_ref[:, :, :, 0:half_sr] + lo_ref[...]
        o_ref[:, :, :, half_sr:n_sr] = a_ref[:, :, :, half_sr:n_sr] + hi_ref[...]

    return pl.pallas_call(
        add_body,
        grid=(channels // c_blk, batch),
        in_specs=[
            pl.BlockSpec((1, c_blk, n, n_sr), lambda c, b: (b, c, 0, 0)),
            pl.BlockSpec((c_blk, n, half_sr), lambda c, b: (c, 0, 0)),
            pl.BlockSpec((c_blk, n, half_sr), lambda c, b: (c, 0, 0)),
        ],
        out_specs=pl.BlockSpec((1, c_blk, n, n_sr), lambda c, b: (b, c, 0, 0)),
        out_shape=jax.ShapeDtypeStruct(attn.shape, attn.dtype),
    )(attn, bias_lo, bias_hi)


def kernel(attn, relative_position_bias_table, relative_position_index):
    batch, channels, n, n_sr = attn.shape
    table_flat = relative_position_bias_table.T.reshape(-1)
    idx_flat = relative_position_index.reshape(-1).astype(jnp.int32)
    bias_lo, bias_hi = _sc_gather_bias(table_flat, idx_flat, channels, n, n_sr)
    return _tc_add(attn, bias_lo, bias_hi)


# speedup vs baseline: 8.8399x; 1.1157x over previous
import functools

import jax
import jax.numpy as jnp
from jax import lax
from jax.experimental import pallas as pl
from jax.experimental.pallas import tpu as pltpu
from jax.experimental.pallas import tpu_sc as plsc

_NUM_CORES = 2
_NUM_SUBCORES = 16
_NUM_WORKERS = _NUM_CORES * _NUM_SUBCORES
_LANES = 16


def _sc_gather_bias(table_flat, idx_flat, num_channels, n, n_sr):
    rows_per_worker = n // _NUM_WORKERS
    row_chunk = rows_per_worker // 2
    half_sr = n_sr // 2
    groups = n_sr // _LANES
    tab_words = table_flat.shape[0]
    table_rows = tab_words // num_channels
    mesh = plsc.VectorSubcoreMesh(core_axis_name="c", subcore_axis_name="s")

    plane = jax.ShapeDtypeStruct((num_channels, n, half_sr), jnp.float32)

    @functools.partial(
        pl.kernel,
        mesh=mesh,
        compiler_params=pltpu.CompilerParams(needs_layout_passes=False),
        out_type=(plane, plane),
        scratch_types=[
            pltpu.VMEM((rows_per_worker * n_sr,), jnp.int32),
            pltpu.VMEM((tab_words,), jnp.float32),
            pltpu.VMEM((num_channels, row_chunk, half_sr), jnp.float32),
            pltpu.VMEM((num_channels, row_chunk, half_sr), jnp.float32),
        ],
    )
    def gather_kernel(tab_hbm, idx_hbm, lo_hbm, hi_hbm, idx_v, tab_v, lo_v, hi_v):
        wid = lax.axis_index("s") * _NUM_CORES + lax.axis_index("c")
        row0 = wid * rows_per_worker
        pltpu.sync_copy(idx_hbm.at[pl.ds(row0 * n_sr, rows_per_worker * n_sr)], idx_v)
        pltpu.sync_copy(tab_hbm, tab_v)
        for chunk in range(2):
            crow = chunk * row_chunk

            @plsc.parallel_loop(0, row_chunk, unroll=2)
            def row_body(r, crow=crow):
                flat = (crow + r) * n_sr
                for k in range(groups):
                    iv = idx_v[pl.ds(flat + k * _LANES, _LANES)]
                    dst = lo_v if k < groups // 2 else hi_v
                    col = (k % (groups // 2)) * _LANES
                    vals = [
                        plsc.load_gather(tab_v, [iv + c * table_rows])
                        for c in range(num_channels)
                    ]
                    for c in range(num_channels):
                        dst[c, r, pl.ds(col, _LANES)] = vals[c]
            for c in range(num_channels):
                pltpu.sync_copy(
                    lo_v.at[c], lo_hbm.at[c, pl.ds(row0 + crow, row_chunk)]
                )
                pltpu.sync_copy(
                    hi_v.at[c], hi_hbm.at[c, pl.ds(row0 + crow, row_chunk)]
                )

    return gather_kernel(table_flat, idx_flat)


def _tc_add(attn, bias_lo, bias_hi):
    batch, channels, n, n_sr = attn.shape
    half_sr = n_sr // 2
    c_blk = 2

    def add_body(a_ref, lo_ref, hi_ref, o_ref):
        o_ref[:, :, :, 0:half_sr] = a_ref[:, :, :, 0:half_sr] + lo_ref[...]
        o_ref[:, :, :, half_sr:n_sr] = a_ref[:, :, :, half_sr:n_sr] + hi_ref[...]

    return pl.pallas_call(
        add_body,
        grid=(channels // c_blk, batch),
        in_specs=[
            pl.BlockSpec((1, c_blk, n, n_sr), lambda c, b: (b, c, 0, 0)),
            pl.BlockSpec((c_blk, n, half_sr), lambda c, b: (c, 0, 0)),
            pl.BlockSpec((c_blk, n, half_sr), lambda c, b: (c, 0, 0)),
        ],
        out_specs=pl.BlockSpec((1, c_blk, n, n_sr), lambda c, b: (b, c, 0, 0)),
        out_shape=jax.ShapeDtypeStruct(attn.shape, attn.dtype),
    )(attn, bias_lo, bias_hi)


def kernel(attn, relative_position_bias_table, relative_position_index):
    batch, channels, n, n_sr = attn.shape
    table_flat = relative_position_bias_table.T.reshape(-1)
    idx_flat = relative_position_index.reshape(-1).astype(jnp.int32)
    bias_lo, bias_hi = _sc_gather_bias(table_flat, idx_flat, channels, n, n_sr)
    return _tc_add(attn, bias_lo, bias_hi)


# speedup vs baseline: 9.4161x; 1.0652x over previous
import functools

import jax
import jax.numpy as jnp
from jax import lax
from jax.experimental import pallas as pl
from jax.experimental.pallas import tpu as pltpu
from jax.experimental.pallas import tpu_sc as plsc

_NUM_CORES = 2
_NUM_SUBCORES = 16
_NUM_WORKERS = _NUM_CORES * _NUM_SUBCORES
_LANES = 16


def _sc_gather_bias(table_flat, idx_flat, num_channels, n, n_sr):
    rows_per_worker = n // _NUM_WORKERS
    row_chunk = rows_per_worker // 2
    half_sr = n_sr // 2
    groups = n_sr // _LANES
    tab_words = table_flat.shape[0]
    table_rows = tab_words // num_channels
    mesh = plsc.VectorSubcoreMesh(core_axis_name="c", subcore_axis_name="s")

    plane = jax.ShapeDtypeStruct((num_channels, n, half_sr), jnp.float32)

    @functools.partial(
        pl.kernel,
        mesh=mesh,
        compiler_params=pltpu.CompilerParams(needs_layout_passes=False),
        out_type=(plane, plane),
        scratch_types=[
            pltpu.VMEM((rows_per_worker * n_sr,), jnp.int32),
            pltpu.VMEM((tab_words,), jnp.float32),
            pltpu.VMEM((num_channels, row_chunk, half_sr), jnp.float32),
            pltpu.VMEM((num_channels, row_chunk, half_sr), jnp.float32),
        ],
    )
    def gather_kernel(tab_hbm, idx_hbm, lo_hbm, hi_hbm, idx_v, tab_v, lo_v, hi_v):
        wid = lax.axis_index("s") * _NUM_CORES + lax.axis_index("c")
        row0 = wid * rows_per_worker
        pltpu.sync_copy(idx_hbm.at[pl.ds(row0 * n_sr, rows_per_worker * n_sr)], idx_v)
        pltpu.sync_copy(tab_hbm, tab_v)
        for chunk in range(2):
            crow = chunk * row_chunk

            @plsc.parallel_loop(0, row_chunk, unroll=2)
            def row_body(r, crow=crow):
                flat = (crow + r) * n_sr
                for k in range(groups):
                    iv = idx_v[pl.ds(flat + k * _LANES, _LANES)]
                    dst = lo_v if k < groups // 2 else hi_v
                    col = (k % (groups // 2)) * _LANES
                    vals = [
                        plsc.load_gather(tab_v, [iv + c * table_rows])
                        for c in range(num_channels)
                    ]
                    for c in range(num_channels):
                        dst[c, r, pl.ds(col, _LANES)] = vals[c]
            for c in range(num_channels):
                pltpu.sync_copy(
                    lo_v.at[c], lo_hbm.at[c, pl.ds(row0 + crow, row_chunk)]
                )
                pltpu.sync_copy(
                    hi_v.at[c], hi_hbm.at[c, pl.ds(row0 + crow, row_chunk)]
                )

    return gather_kernel(table_flat, idx_flat)


def _tc_add(attn, bias_lo, bias_hi):
    batch, channels, n, n_sr = attn.shape
    half_sr = n_sr // 2
    c_blk = 4

    def add_body(a_ref, lo_ref, hi_ref, o_ref):
        o_ref[:, :, :, 0:half_sr] = a_ref[:, :, :, 0:half_sr] + lo_ref[...]
        o_ref[:, :, :, half_sr:n_sr] = a_ref[:, :, :, half_sr:n_sr] + hi_ref[...]

    return pl.pallas_call(
        add_body,
        grid=(channels // c_blk, batch),
        in_specs=[
            pl.BlockSpec((1, c_blk, n, n_sr), lambda c, b: (b, c, 0, 0)),
            pl.BlockSpec((c_blk, n, half_sr), lambda c, b: (c, 0, 0)),
            pl.BlockSpec((c_blk, n, half_sr), lambda c, b: (c, 0, 0)),
        ],
        out_specs=pl.BlockSpec((1, c_blk, n, n_sr), lambda c, b: (b, c, 0, 0)),
        out_shape=jax.ShapeDtypeStruct(attn.shape, attn.dtype),
    )(attn, bias_lo, bias_hi)


def kernel(attn, relative_position_bias_table, relative_position_index):
    batch, channels, n, n_sr = attn.shape
    table_flat = relative_position_bias_table.T.reshape(-1)
    idx_flat = relative_position_index.reshape(-1).astype(jnp.int32)
    bias_lo, bias_hi = _sc_gather_bias(table_flat, idx_flat, channels, n, n_sr)
    return _tc_add(attn, bias_lo, bias_hi)


# speedup vs baseline: 9.5611x; 1.0154x over previous
import functools

import jax
import jax.numpy as jnp
from jax import lax
from jax.experimental import pallas as pl
from jax.experimental.pallas import tpu as pltpu
from jax.experimental.pallas import tpu_sc as plsc

_NUM_CORES = 2
_NUM_SUBCORES = 16
_NUM_WORKERS = _NUM_CORES * _NUM_SUBCORES
_LANES = 16


def _sc_gather_bias(table_flat, idx_flat, num_channels, n, n_sr):
    rows_per_worker = n // _NUM_WORKERS
    row_chunk = rows_per_worker // 2
    half_sr = n_sr // 2
    groups = n_sr // _LANES
    tab_words = table_flat.shape[0]
    table_rows = tab_words // num_channels
    mesh = plsc.VectorSubcoreMesh(core_axis_name="c", subcore_axis_name="s")

    plane = jax.ShapeDtypeStruct((num_channels, n, half_sr), jnp.float32)

    @functools.partial(
        pl.kernel,
        mesh=mesh,
        compiler_params=pltpu.CompilerParams(needs_layout_passes=False),
        out_type=(plane, plane),
        scratch_types=[
            pltpu.VMEM((rows_per_worker * n_sr,), jnp.int32),
            pltpu.VMEM((tab_words,), jnp.float32),
            pltpu.VMEM((num_channels, row_chunk, half_sr), jnp.float32),
            pltpu.VMEM((num_channels, row_chunk, half_sr), jnp.float32),
        ],
    )
    def gather_kernel(tab_hbm, idx_hbm, lo_hbm, hi_hbm, idx_v, tab_v, lo_v, hi_v):
        wid = lax.axis_index("s") * _NUM_CORES + lax.axis_index("c")
        row0 = wid * rows_per_worker
        pltpu.sync_copy(idx_hbm.at[pl.ds(row0 * n_sr, rows_per_worker * n_sr)], idx_v)
        pltpu.sync_copy(tab_hbm, tab_v)
        for chunk in range(2):
            crow = chunk * row_chunk

            @plsc.parallel_loop(0, row_chunk, unroll=2)
            def row_body(r, crow=crow):
                flat = (crow + r) * n_sr
                for k in range(groups):
                    iv = idx_v[pl.ds(flat + k * _LANES, _LANES)]
                    dst = lo_v if k < groups // 2 else hi_v
                    col = (k % (groups // 2)) * _LANES
                    vals = [
                        plsc.load_gather(tab_v, [iv + c * table_rows])
                        for c in range(num_channels)
                    ]
                    for c in range(num_channels):
                        dst[c, r, pl.ds(col, _LANES)] = vals[c]
            for c in range(num_channels):
                pltpu.sync_copy(
                    lo_v.at[c], lo_hbm.at[c, pl.ds(row0 + crow, row_chunk)]
                )
                pltpu.sync_copy(
                    hi_v.at[c], hi_hbm.at[c, pl.ds(row0 + crow, row_chunk)]
                )

    return gather_kernel(table_flat, idx_flat)


def _tc_add(attn, bias_lo, bias_hi):
    batch, channels, n, n_sr = attn.shape
    half_sr = n_sr // 2
    c_blk = 6

    def add_body(a_ref, lo_ref, hi_ref, o_ref):
        o_ref[:, :, :, 0:half_sr] = a_ref[:, :, :, 0:half_sr] + lo_ref[...]
        o_ref[:, :, :, half_sr:n_sr] = a_ref[:, :, :, half_sr:n_sr] + hi_ref[...]

    return pl.pallas_call(
        add_body,
        grid=(channels // c_blk, batch),
        in_specs=[
            pl.BlockSpec((1, c_blk, n, n_sr), lambda c, b: (b, c, 0, 0)),
            pl.BlockSpec((c_blk, n, half_sr), lambda c, b: (c, 0, 0)),
            pl.BlockSpec((c_blk, n, half_sr), lambda c, b: (c, 0, 0)),
        ],
        out_specs=pl.BlockSpec((1, c_blk, n, n_sr), lambda c, b: (b, c, 0, 0)),
        out_shape=jax.ShapeDtypeStruct(attn.shape, attn.dtype),
    )(attn, bias_lo, bias_hi)


def kernel(attn, relative_position_bias_table, relative_position_index):
    batch, channels, n, n_sr = attn.shape
    table_flat = relative_position_bias_table.T.reshape(-1)
    idx_flat = relative_position_index.reshape(-1).astype(jnp.int32)
    bias_lo, bias_hi = _sc_gather_bias(table_flat, idx_flat, channels, n, n_sr)
    return _tc_add(attn, bias_lo, bias_hi)
